# baseline (speedup 1.0000x reference)
import functools

import jax
import jax.numpy as jnp
from jax import lax
from jax.experimental import pallas as pl
from jax.experimental.pallas import tpu as pltpu
from jax.experimental.pallas import tpu_sc as plsc

VOCAB = 2548
DIM = 1000
BATCH = 4096

ROWS = 8
NCHUNKS = DIM // ROWS
LANES = 16


def _make_embedding_kernel():
    info = plsc.get_sparse_core_info()
    num_cores, num_subcores = info.num_cores, info.num_subcores
    num_workers = num_cores * num_subcores
    max_chunks = -(-NCHUNKS // num_workers)

    mesh = plsc.VectorSubcoreMesh(core_axis_name="c", subcore_axis_name="s")

    @functools.partial(
        pl.kernel,
        mesh=mesh,
        out_type=jax.ShapeDtypeStruct((DIM, BATCH), jnp.float32),
        scratch_types=[
            pltpu.VMEM((BATCH,), jnp.int32),
            [pltpu.VMEM((ROWS, VOCAB), jnp.float32) for _ in range(2)],
            [pltpu.VMEM((ROWS, BATCH), jnp.float32) for _ in range(2)],
            pltpu.SemaphoreType.DMA,
            [pltpu.SemaphoreType.DMA for _ in range(2)],
            [pltpu.SemaphoreType.DMA for _ in range(2)],
        ],
        compiler_params=pltpu.CompilerParams(needs_layout_passes=False),
    )
    def emb(x_hbm, wt_hbm, out_hbm, idx_v, in_ts, out_fs, xsem, gsems, wsems):
        wid = lax.axis_index("s") * num_cores + lax.axis_index("c")
        pltpu.async_copy(x_hbm, idx_v, xsem)

        rows_splat = [jnp.full((LANES,), r, jnp.int32) for r in range(ROWS)]

        def cval(k):
            return wid + k * num_workers

        def in_slab(k):
            return wt_hbm.at[pl.ds(cval(k) * ROWS, ROWS)]

        def out_slab(k):
            return out_hbm.at[pl.ds(cval(k) * ROWS, ROWS)]

        pltpu.async_copy(in_slab(0), in_ts[0], gsems[0])
        pltpu.make_async_copy(x_hbm, idx_v, xsem).wait()

        def do_chunk(k):
            if k + 1 < max_chunks:

                def prefetch():
                    pltpu.async_copy(
                        in_slab(k + 1), in_ts[(k + 1) % 2], gsems[(k + 1) % 2]
                    )

                if k + 2 == max_chunks:
                    pl.when(cval(k + 1) < NCHUNKS)(prefetch)
                else:
                    prefetch()

            pltpu.make_async_copy(in_slab(k), in_ts[k % 2], gsems[k % 2]).wait()
            if k >= 2:
                pltpu.make_async_copy(
                    out_fs[k % 2], out_slab(k - 2), wsems[k % 2]
                ).wait()

            in_t = in_ts[k % 2]
            out_f = out_fs[k % 2]
            half = BATCH // (2 * LANES)

            @plsc.parallel_loop(0, half, unroll=4)
            def _gather_lo(j):
                cols = idx_v[pl.ds(j * LANES, LANES)]
                for r in range(ROWS):
                    v = plsc.load_gather(in_t, [rows_splat[r], cols])
                    out_f[r, pl.ds(j * LANES, LANES)] = v

            pltpu.async_copy(
                out_f.at[:, pl.ds(0, BATCH // 2)],
                out_slab(k).at[:, pl.ds(0, BATCH // 2)],
                wsems[k % 2],
            )

            @plsc.parallel_loop(half, 2 * half, unroll=4)
            def _gather_hi(j):
                cols = idx_v[pl.ds(j * LANES, LANES)]
                for r in range(ROWS):
                    v = plsc.load_gather(in_t, [rows_splat[r], cols])
                    out_f[r, pl.ds(j * LANES, LANES)] = v

            pltpu.async_copy(
                out_f.at[:, pl.ds(BATCH // 2, BATCH // 2)],
                out_slab(k).at[:, pl.ds(BATCH // 2, BATCH // 2)],
                wsems[k % 2],
            )

        for k in range(max_chunks):
            if k == max_chunks - 1:

                def last_chunk(k=k):
                    do_chunk(k)

                pl.when(cval(k) < NCHUNKS)(last_chunk)
            else:
                do_chunk(k)

        for k in range(max(0, max_chunks - 2), max_chunks):

            @pl.when(cval(k) < NCHUNKS)
            def _():
                pltpu.make_async_copy(
                    out_fs[k % 2], out_slab(k), wsems[k % 2]
                ).wait()

    return emb


_emb = _make_embedding_kernel()


def kernel(x, weight):
    out_t = _emb(x.astype(jnp.int32), weight.T)
    return out_t.T


# speedup vs baseline: 1.0210x; 1.0210x over previous
import functools

import jax
import jax.numpy as jnp
from jax import lax
from jax.experimental import pallas as pl
from jax.experimental.pallas import tpu as pltpu
from jax.experimental.pallas import tpu_sc as plsc

VOCAB = 2548
DIM = 1000
BATCH = 4096

ROWS = 8
NCHUNKS = DIM // ROWS
LANES = 16


def _make_embedding_kernel():
    info = plsc.get_sparse_core_info()
    num_cores, num_subcores = info.num_cores, info.num_subcores
    num_workers = num_cores * num_subcores
    max_chunks = -(-NCHUNKS // num_workers)

    mesh = plsc.VectorSubcoreMesh(core_axis_name="c", subcore_axis_name="s")

    @functools.partial(
        pl.kernel,
        mesh=mesh,
        out_type=jax.ShapeDtypeStruct((DIM, BATCH), jnp.float32),
        scratch_types=[
            pltpu.VMEM((BATCH,), jnp.int32),
            [pltpu.VMEM((ROWS, VOCAB), jnp.float32) for _ in range(2)],
            [pltpu.VMEM((ROWS, BATCH), jnp.float32) for _ in range(2)],
            pltpu.SemaphoreType.DMA,
            [pltpu.SemaphoreType.DMA for _ in range(2)],
            [pltpu.SemaphoreType.DMA for _ in range(2)],
        ],
        compiler_params=pltpu.CompilerParams(needs_layout_passes=False),
    )
    def emb(x_hbm, wt_hbm, out_hbm, idx_v, in_ts, out_fs, xsem, gsems, wsems):
        wid = lax.axis_index("s") * num_cores + lax.axis_index("c")
        pltpu.async_copy(x_hbm, idx_v, xsem)

        rows_splat = [jnp.full((LANES,), r, jnp.int32) for r in range(ROWS)]

        def cval(k):
            return wid + k * num_workers

        def in_slab(k):
            return wt_hbm.at[pl.ds(cval(k) * ROWS, ROWS)]

        def out_slab(k):
            return out_hbm.at[pl.ds(cval(k) * ROWS, ROWS)]

        pltpu.async_copy(in_slab(0), in_ts[0], gsems[0])
        pltpu.make_async_copy(x_hbm, idx_v, xsem).wait()

        def do_chunk(k):
            if k + 1 < max_chunks:

                def prefetch():
                    pltpu.async_copy(
                        in_slab(k + 1), in_ts[(k + 1) % 2], gsems[(k + 1) % 2]
                    )

                if k + 2 == max_chunks:
                    pl.when(cval(k + 1) < NCHUNKS)(prefetch)
                else:
                    prefetch()

            pltpu.make_async_copy(in_slab(k), in_ts[k % 2], gsems[k % 2]).wait()
            if k >= 2:
                pltpu.make_async_copy(
                    out_fs[k % 2], out_slab(k - 2), wsems[k % 2]
                ).wait()

            in_t = in_ts[k % 2]
            out_f = out_fs[k % 2]

            @plsc.parallel_loop(0, BATCH // LANES, unroll=4)
            def _gather(j):
                cols = idx_v[pl.ds(j * LANES, LANES)]
                for r in range(ROWS):
                    v = plsc.load_gather(in_t, [rows_splat[r], cols])
                    out_f[r, pl.ds(j * LANES, LANES)] = v

            pltpu.async_copy(out_f, out_slab(k), wsems[k % 2])

        for k in range(max_chunks):
            if k == max_chunks - 1:

                def last_chunk(k=k):
                    do_chunk(k)

                pl.when(cval(k) < NCHUNKS)(last_chunk)
            else:
                do_chunk(k)

        for k in range(max_chunks):
            if k + 2 < max_chunks - 1:
                continue
            if k + 2 == max_chunks - 1:
                cond = (cval(k) < NCHUNKS) & (cval(k + 2) >= NCHUNKS)
            else:
                cond = cval(k) < NCHUNKS

            def drain(k=k):
                pltpu.make_async_copy(
                    out_fs[k % 2], out_slab(k), wsems[k % 2]
                ).wait()

            pl.when(cond)(drain)

    return emb


_emb = _make_embedding_kernel()


def kernel(x, weight):
    out_t = _emb(x.astype(jnp.int32), weight.T)
    return out_t.T
